# Initial kernel scaffold; baseline (speedup 1.0000x reference)
import jax
import jax.numpy as jnp
from jax.experimental import pallas as pl


def kernel(x, conv1_w, conv1_b, conv2_w, conv2_b, lin1_w, lin1_b, lin2_w, lin2_b, lin3_w, lin3_b):
    raise NotImplementedError("write your pallas kernel here")


# speedup vs baseline: 5.9471x; 5.9471x over previous
import jax
import jax.numpy as jnp
from jax.experimental import pallas as pl
from jax.experimental.pallas import tpu as pltpu


def _round_up(v, m):
    return (v + m - 1) // m * m


_NB = 128


def _lenet_kernel(x_ref,
                  m1, b1, m2, b2,
                  f1k, f1b, f2w, f2b, f3w, f3b,
                  out_ref):
    f32 = jnp.float32
    nb = _NB
    xin = x_ref[0]

    acc = jnp.dot(m1[0], xin[:, : 28 * nb], preferred_element_type=f32)
    for j in range(1, 5):
        acc = acc + jnp.dot(m1[j], xin[:, j * nb:(j + 28) * nb],
                            preferred_element_type=f32)
    a1 = jnp.maximum(acc + b1[...], 0.0)

    px = jnp.concatenate(
        [jnp.maximum(a1[:, 2 * k * nb:(2 * k + 1) * nb],
                     a1[:, (2 * k + 1) * nb:(2 * k + 2) * nb])
         for k in range(14)], axis=1)
    myf = jnp.maximum(px[:167], px[1:168])

    acc2 = jnp.dot(m2[0], myf[:, : 10 * nb], preferred_element_type=f32)
    for j in range(1, 5):
        acc2 = acc2 + jnp.dot(m2[j], myf[:, j * nb:(j + 10) * nb],
                              preferred_element_type=f32)
    a2 = jnp.maximum(acc2 + b2[...], 0.0)

    pz = jnp.concatenate(
        [jnp.maximum(a2[:, 2 * k * nb:(2 * k + 1) * nb],
                     a2[:, (2 * k + 1) * nb:(2 * k + 2) * nb])
         for k in range(5)], axis=1)
    mz = jnp.maximum(pz[:159], pz[1:160])

    h1 = jnp.dot(f1k[0], mz[:, :nb], preferred_element_type=f32)
    for k in range(1, 5):
        h1 = h1 + jnp.dot(f1k[k], mz[:, k * nb:(k + 1) * nb],
                          preferred_element_type=f32)
    h1 = jnp.maximum(h1 + f1b[...], 0.0)
    h2 = jnp.maximum(jnp.dot(f2w[...], h1, preferred_element_type=f32)
                     + f2b[...], 0.0)
    out_ref[...] = (jnp.dot(f3w[...], h2, preferred_element_type=f32)
                    + f3b[...])


@jax.jit
def _lenet_forward(x, conv1_w, conv1_b, conv2_w, conv2_b,
                   lin1_w, lin1_b, lin2_w, lin2_b, lin3_w, lin3_b):
    f32 = jnp.float32
    B = x.shape[0]
    nb = _NB
    bp = _round_up(B, nb)
    nblk = bp // nb

    xr = x.astype(f32).reshape(B, 28, 28)
    xpad = jnp.pad(xr, ((0, bp - B), (2, 2), (2, 2)))
    xin = (xpad.reshape(nblk, nb, 32, 32)
           .transpose(0, 2, 3, 1)
           .reshape(nblk, 32, 32 * nb))

    e1 = jnp.stack([jnp.eye(28, 32, k=i, dtype=f32) for i in range(5)])
    w1 = conv1_w.reshape(6, 5, 5).astype(f32)
    m1 = jnp.einsum('oij,iyk->joyk', w1, e1).reshape(5, 168, 32)

    i_ = jnp.arange(5)[:, None, None]
    y_ = jnp.arange(10)[None, :, None]
    k_ = jnp.arange(28)[None, None, :]
    e2 = (k_ == 2 * (y_ + i_)).astype(f32)
    w2 = conv2_w.astype(f32)
    m2 = (jnp.einsum('ocij,iyk->joyck', w2, e2)
          .reshape(5, 160, 168)[:, :, :167])

    b1r = jnp.repeat(conv1_b.astype(f32), 28).reshape(168, 1)
    b2r = jnp.repeat(conv2_b.astype(f32), 10).reshape(160, 1)

    kk, oo, yy = jnp.meshgrid(jnp.arange(5), jnp.arange(16), jnp.arange(5),
                              indexing='ij')
    kk, oo, yy = kk.ravel(), oo.ravel(), yy.ravel()
    f1w = lin1_w.astype(f32)
    f1k = (jnp.zeros((5, 100, 159), f32)
           .at[kk, :, oo * 10 + 2 * yy]
           .set(f1w[:, oo * 25 + yy * 5 + kk].T))

    f1b = lin1_b.astype(f32).reshape(100, 1)
    f2w = lin2_w.astype(f32)
    f2b = lin2_b.astype(f32).reshape(50, 1)
    f3w = lin3_w.astype(f32)
    f3b = lin3_b.astype(f32).reshape(10, 1)

    def resident(shape):
        n = len(shape)
        return pl.BlockSpec(shape, lambda b: (0,) * n)

    out = pl.pallas_call(
        _lenet_kernel,
        out_shape=jax.ShapeDtypeStruct((10, bp), f32),
        grid=(nblk,),
        in_specs=[
            pl.BlockSpec((1, 32, 32 * nb), lambda b: (b, 0, 0)),
            resident((5, 168, 32)), resident((168, 1)),
            resident((5, 160, 167)), resident((160, 1)),
            resident((5, 100, 159)), resident((100, 1)),
            resident((50, 100)), resident((50, 1)),
            resident((10, 50)), resident((10, 1)),
        ],
        out_specs=pl.BlockSpec((10, nb), lambda b: (0, b)),
        compiler_params=pltpu.CompilerParams(
            dimension_semantics=("parallel",),
            vmem_limit_bytes=64 * 1024 * 1024,
        ),
    )(xin, m1, b1r, m2, b2r, f1k, f1b, f2w, f2b, f3w, f3b)

    return out[:, :B].T


def kernel(x, conv1_w, conv1_b, conv2_w, conv2_b,
           lin1_w, lin1_b, lin2_w, lin2_b, lin3_w, lin3_b):
    return _lenet_forward(x, conv1_w, conv1_b, conv2_w, conv2_b,
                          lin1_w, lin1_b, lin2_w, lin2_b, lin3_w, lin3_b)
